# baseline (speedup 1.0000x reference)
import functools

import numpy as np
import jax
import jax.numpy as jnp
from jax import lax
from jax.experimental import pallas as pl
from jax.experimental.pallas import tpu as pltpu
from jax.experimental.pallas import tpu_sc as plsc

B = 4096
L = 50
LP1 = L + 1
DIM = 512
VOCAB = 28
MAX_CHARGE = 10

VS = 32
CB = L * VS
TROWS = CB + 16

NC = 2
NS = 16
NW = NC * NS

S = 2
BS = B // S
RPS = BS * LP1
C = 64
NCH = RPS // NW // C

DB = 16
NBLK = BS // DB


def _positional_encoding_np(length, d_model):
    pos = np.arange(length, dtype=np.float32)[:, None]
    i = np.arange(d_model // 2, dtype=np.float32)[None, :]
    angle = pos / np.power(10000.0, (2.0 * i) / d_model)
    pe = np.zeros((length, d_model), dtype=np.float32)
    pe[:, 0::2] = np.sin(angle)
    pe[:, 1::2] = np.cos(angle)
    return pe


_PE = _positional_encoding_np(L, DIM)


def _prep_body(pe_ref, aa_ref, ch_ref, tok_ref, chg_ref, t3_ref, tc_ref, idx_ref):
    aa = aa_ref[...]
    row = lax.broadcasted_iota(jnp.int32, (VOCAB, DIM), 0)
    aa_z = jnp.where(row == VOCAB - 1, 0.0, aa)
    aa_p = jnp.concatenate(
        [aa_z, jnp.zeros((VS - VOCAB, DIM), jnp.float32)], axis=0)
    pe = pe_ref[...]
    t3_ref[...] = pe[:, None, :] + aa_p[None, :, :]
    ch = ch_ref[...]
    tc_ref[...] = jnp.concatenate(
        [ch, jnp.zeros((TROWS - CB - MAX_CHARGE, DIM), jnp.float32)], axis=0)
    pos_off = lax.broadcasted_iota(jnp.int32, (B, L), 1) * VS
    idx_ref[...] = jnp.concatenate(
        [chg_ref[...] + CB, tok_ref[...] + pos_off], axis=1)


def _prep(tokens, charges):
    return pl.pallas_call(
        _prep_body,
        out_shape=[
            jax.ShapeDtypeStruct((L, VS, DIM), jnp.float32),
            jax.ShapeDtypeStruct((TROWS - CB, DIM), jnp.float32),
            jax.ShapeDtypeStruct((B, LP1), jnp.int32),
        ],
    )


def _gather_body(tab_hbm, idx_hbm, out_hbm, idx_v, b0, b1, g0, g1, s0, s1):
    bufs, gsems, ssems = (b0, b1), (g0, g1), (s0, s1)
    wid = lax.axis_index("s") * NC + lax.axis_index("c")
    cbase = wid * NCH
    pltpu.sync_copy(idx_hbm.at[wid], idx_v)

    def g_start(k, p):
        pltpu.async_copy(tab_hbm.at[idx_v.at[k]], bufs[p], gsems[p])

    def g_wait(p):
        pltpu.make_async_copy(tab_hbm.at[pl.ds(0, C)], bufs[p],
                              gsems[p]).wait()

    def s_start(k, p):
        pltpu.async_copy(bufs[p], out_hbm.at[pl.ds((cbase + k) * C, C)],
                         ssems[p])

    def s_wait(p):
        pltpu.make_async_copy(bufs[p], out_hbm.at[pl.ds(0, C)],
                              ssems[p]).wait()

    g_start(0, 0)
    g_wait(0); s_start(0, 0); g_start(1, 1)

    def group(g, carry):
        for a in range(2):
            k = 2 * g + 1 + a
            p = 1 - a
            p1 = 1 - p
            g_wait(p)
            s_start(k, p)
            s_wait(p1)
            g_start(k + 1, p1)
        return carry

    lax.fori_loop(0, (NCH - 3) // 2, group, 0)

    g_wait(1); s_start(NCH - 2, 1); s_wait(0); g_start(NCH - 1, 0)
    g_wait(0); s_start(NCH - 1, 0); s_wait(1)
    s_wait(0)


_MESH = plsc.VectorSubcoreMesh(
    core_axis_name="c", subcore_axis_name="s", num_cores=NC, num_subcores=NS)

_sc_gather = functools.partial(
    pl.kernel,
    out_type=jax.ShapeDtypeStruct((RPS, DIM), jnp.float32),
    mesh=_MESH,
    scratch_types=[
        pltpu.VMEM((NCH, C), jnp.int32),
        pltpu.VMEM((C, DIM), jnp.float32),
        pltpu.VMEM((C, DIM), jnp.float32),
        pltpu.SemaphoreType.DMA,
        pltpu.SemaphoreType.DMA,
        pltpu.SemaphoreType.DMA,
        pltpu.SemaphoreType.DMA,
    ],
)(_gather_body)


def _retile_body(in_ref, out_ref):
    out_ref[...] = in_ref[...].reshape(DB, LP1, DIM)


def _retile(flat):
    return pl.pallas_call(
        _retile_body,
        grid=(NBLK,),
        in_specs=[pl.BlockSpec((DB * LP1, DIM), lambda i: (i, 0))],
        out_specs=pl.BlockSpec((DB, LP1, DIM), lambda i: (i, 0, 0)),
        out_shape=jax.ShapeDtypeStruct((BS, LP1, DIM), jnp.float32),
    )(flat)


def kernel(tokens, charges, aa_table, charge_table):
    tokens = tokens.astype(jnp.int32)
    charges = charges.astype(jnp.int32).reshape(B, 1)
    t3, tc, idx = _prep(tokens, charges)(
        jnp.asarray(_PE), aa_table, charge_table, tokens, charges)
    table = jnp.concatenate([t3.reshape(CB, DIM), tc], axis=0)
    idx4 = idx.reshape(S, NW, NCH, C)
    parts = []
    for s in range(S):
        flat = _sc_gather(table, idx4[s])
        parts.append(_retile(flat))
    return jnp.concatenate(parts, axis=0)


# speedup vs baseline: 1.2915x; 1.2915x over previous
import functools

import numpy as np
import jax
import jax.numpy as jnp
from jax import lax
from jax.experimental import pallas as pl
from jax.experimental.pallas import tpu as pltpu
from jax.experimental.pallas import tpu_sc as plsc

B = 4096
L = 50
LP1 = L + 1
DIM = 512
VOCAB = 28
MAX_CHARGE = 10

VS = 32
CB = L * VS
TROWS = CB + 16

R = B * LP1
NC = 2
NS = 16
NW = NC * NS
RPW = R // NW
C = 64
NCH = RPW // C
NB = 3
NG = NCH // NB


def _positional_encoding_np(length, d_model):
    pos = np.arange(length, dtype=np.float32)[:, None]
    i = np.arange(d_model // 2, dtype=np.float32)[None, :]
    angle = pos / np.power(10000.0, (2.0 * i) / d_model)
    pe = np.zeros((length, d_model), dtype=np.float32)
    pe[:, 0::2] = np.sin(angle)
    pe[:, 1::2] = np.cos(angle)
    return pe


_PE = _positional_encoding_np(L, DIM)


def _prep_body(pe_ref, aa_ref, ch_ref, tok_ref, chg_ref, t3_ref, tc_ref, idx_ref):
    aa = aa_ref[...]
    row = lax.broadcasted_iota(jnp.int32, (VOCAB, DIM), 0)
    aa_z = jnp.where(row == VOCAB - 1, 0.0, aa)
    aa_p = jnp.concatenate(
        [aa_z, jnp.zeros((VS - VOCAB, DIM), jnp.float32)], axis=0)
    pe = pe_ref[...]
    t3_ref[...] = pe[:, None, :] + aa_p[None, :, :]
    ch = ch_ref[...]
    tc_ref[...] = jnp.concatenate(
        [ch, jnp.zeros((TROWS - CB - MAX_CHARGE, DIM), jnp.float32)], axis=0)
    pos_off = lax.broadcasted_iota(jnp.int32, (B, L), 1) * VS
    idx_ref[...] = jnp.concatenate(
        [chg_ref[...] + CB, tok_ref[...] + pos_off], axis=1)


def _prep(tokens, charges):
    return pl.pallas_call(
        _prep_body,
        out_shape=[
            jax.ShapeDtypeStruct((L, VS, DIM), jnp.float32),
            jax.ShapeDtypeStruct((TROWS - CB, DIM), jnp.float32),
            jax.ShapeDtypeStruct((B, LP1), jnp.int32),
        ],
    )


def _gather_body(tab_hbm, idx_hbm, out_hbm, idx_v,
                 b0, b1, b2, g0, g1, g2, s0, s1, s2):
    bufs, gsems, ssems = (b0, b1, b2), (g0, g1, g2), (s0, s1, s2)
    wid = lax.axis_index("s") * NC + lax.axis_index("c")
    cbase = wid * NCH
    pltpu.sync_copy(idx_hbm.at[wid], idx_v)

    def g_start(k, p):
        pltpu.async_copy(tab_hbm.at[idx_v.at[k]], bufs[p], gsems[p])

    def g_wait(p):
        pltpu.make_async_copy(tab_hbm.at[pl.ds(0, C)], bufs[p], gsems[p]).wait()

    def s_start(k, p):
        pltpu.async_copy(bufs[p], out_hbm.at[pl.ds((cbase + k) * C, C)],
                         ssems[p])

    def s_wait(p):
        pltpu.make_async_copy(bufs[p], out_hbm.at[pl.ds(0, C)],
                              ssems[p]).wait()

    g_start(0, 0)
    g_wait(0); s_start(0, 0); g_start(1, 1)
    g_wait(1); s_start(1, 1); g_start(2, 2)
    g_wait(2); s_start(2, 2); s_wait(0); g_start(3, 0)

    def group(g, carry):
        for s in range(NB):
            k = g * NB + s
            p, p1 = s, (s + 1) % NB
            g_wait(p)
            s_start(k, p)
            s_wait(p1)
            g_start(k + 1, p1)
        return carry

    lax.fori_loop(1, NG - 1, group, 0)

    k0 = (NG - 1) * NB
    g_wait(0); s_start(k0, 0); s_wait(1); g_start(k0 + 1, 1)
    g_wait(1); s_start(k0 + 1, 1); s_wait(2); g_start(k0 + 2, 2)
    g_wait(2); s_start(k0 + 2, 2)
    s_wait(0); s_wait(1); s_wait(2)


_MESH = plsc.VectorSubcoreMesh(
    core_axis_name="c", subcore_axis_name="s", num_cores=NC, num_subcores=NS)

_sc_gather = functools.partial(
    pl.kernel,
    out_type=jax.ShapeDtypeStruct((R, DIM), jnp.float32),
    mesh=_MESH,
    scratch_types=[
        pltpu.VMEM((NCH, C), jnp.int32),
        pltpu.VMEM((C, DIM), jnp.float32),
        pltpu.VMEM((C, DIM), jnp.float32),
        pltpu.VMEM((C, DIM), jnp.float32),
        pltpu.SemaphoreType.DMA,
        pltpu.SemaphoreType.DMA,
        pltpu.SemaphoreType.DMA,
        pltpu.SemaphoreType.DMA,
        pltpu.SemaphoreType.DMA,
        pltpu.SemaphoreType.DMA,
    ],
)(_gather_body)


def kernel(tokens, charges, aa_table, charge_table):
    tokens = tokens.astype(jnp.int32)
    charges = charges.astype(jnp.int32).reshape(B, 1)
    t3, tc, idx = _prep(tokens, charges)(
        jnp.asarray(_PE), aa_table, charge_table, tokens, charges)
    table = jnp.concatenate([t3.reshape(CB, DIM), tc], axis=0)
    idx2 = idx.reshape(NW, NCH, C)
    out = _sc_gather(table, idx2)
    return out.reshape(B, LP1, DIM)


# speedup vs baseline: 3.2202x; 2.4934x over previous
import functools

import numpy as np
import jax
import jax.numpy as jnp
from jax import lax
from jax.experimental import pallas as pl
from jax.experimental.pallas import tpu as pltpu
from jax.experimental.pallas import tpu_sc as plsc

B = 4096
L = 50
LP1 = L + 1
DIM = 512
VOCAB = 28
MAX_CHARGE = 10

VS = 32
CB = L * VS
TROWS = CB + 16

R = B * LP1
NC = 2
NS = 16
NW = NC * NS
RPW = R // NW
C = 64
NCH = RPW // C
NB = 3
NG = NCH // NB


def _positional_encoding_np(length, d_model):
    pos = np.arange(length, dtype=np.float32)[:, None]
    i = np.arange(d_model // 2, dtype=np.float32)[None, :]
    angle = pos / np.power(10000.0, (2.0 * i) / d_model)
    pe = np.zeros((length, d_model), dtype=np.float32)
    pe[:, 0::2] = np.sin(angle)
    pe[:, 1::2] = np.cos(angle)
    return pe


_PE = _positional_encoding_np(L, DIM)


def _prep_body(pe_ref, aa_ref, ch_ref, tok_ref, chg_ref, t3_ref, tc_ref, idx_ref):
    aa = aa_ref[...]
    row = lax.broadcasted_iota(jnp.int32, (VOCAB, DIM), 0)
    aa_z = jnp.where(row == VOCAB - 1, 0.0, aa)
    aa_p = jnp.concatenate(
        [aa_z, jnp.zeros((VS - VOCAB, DIM), jnp.float32)], axis=0)
    pe = pe_ref[...]
    t3_ref[...] = pe[:, None, :] + aa_p[None, :, :]
    ch = ch_ref[...]
    tc_ref[...] = jnp.concatenate(
        [ch, jnp.zeros((TROWS - CB - MAX_CHARGE, DIM), jnp.float32)], axis=0)
    pos_off = lax.broadcasted_iota(jnp.int32, (L, B), 0) * VS
    idx_ref[...] = jnp.concatenate(
        [chg_ref[...] + CB, tok_ref[...] + pos_off], axis=0)


def _prep(tokens, charges):
    return pl.pallas_call(
        _prep_body,
        out_shape=[
            jax.ShapeDtypeStruct((L, VS, DIM), jnp.float32),
            jax.ShapeDtypeStruct((TROWS - CB, DIM), jnp.float32),
            jax.ShapeDtypeStruct((LP1, B), jnp.int32),
        ],
    )


def _gather_body(tab_hbm, idx_hbm, out_hbm, idx_v,
                 b0, b1, b2, g0, g1, g2, s0, s1, s2):
    bufs, gsems, ssems = (b0, b1, b2), (g0, g1, g2), (s0, s1, s2)
    wid = lax.axis_index("s") * NC + lax.axis_index("c")
    cbase = wid * NCH
    pltpu.sync_copy(idx_hbm.at[wid], idx_v)

    def g_start(k, p):
        pltpu.async_copy(tab_hbm.at[idx_v.at[k]], bufs[p], gsems[p])

    def g_wait(p):
        pltpu.make_async_copy(tab_hbm.at[pl.ds(0, C)], bufs[p], gsems[p]).wait()

    def s_start(k, p):
        pltpu.async_copy(bufs[p], out_hbm.at[pl.ds((cbase + k) * C, C)],
                         ssems[p])

    def s_wait(p):
        pltpu.make_async_copy(bufs[p], out_hbm.at[pl.ds(0, C)],
                              ssems[p]).wait()

    g_start(0, 0)
    g_wait(0); s_start(0, 0); g_start(1, 1)
    g_wait(1); s_start(1, 1); g_start(2, 2)
    g_wait(2); s_start(2, 2); s_wait(0); g_start(3, 0)

    def group(g, carry):
        for s in range(NB):
            k = g * NB + s
            p, p1 = s, (s + 1) % NB
            g_wait(p)
            s_start(k, p)
            s_wait(p1)
            g_start(k + 1, p1)
        return carry

    lax.fori_loop(1, NG - 1, group, 0)

    k0 = (NG - 1) * NB
    g_wait(0); s_start(k0, 0); s_wait(1); g_start(k0 + 1, 1)
    g_wait(1); s_start(k0 + 1, 1); s_wait(2); g_start(k0 + 2, 2)
    g_wait(2); s_start(k0 + 2, 2)
    s_wait(0); s_wait(1); s_wait(2)


_MESH = plsc.VectorSubcoreMesh(
    core_axis_name="c", subcore_axis_name="s", num_cores=NC, num_subcores=NS)

_sc_gather = functools.partial(
    pl.kernel,
    out_type=jax.ShapeDtypeStruct((R, DIM), jnp.float32),
    mesh=_MESH,
    scratch_types=[
        pltpu.VMEM((NCH, C), jnp.int32),
        pltpu.VMEM((C, DIM), jnp.float32),
        pltpu.VMEM((C, DIM), jnp.float32),
        pltpu.VMEM((C, DIM), jnp.float32),
        pltpu.SemaphoreType.DMA,
        pltpu.SemaphoreType.DMA,
        pltpu.SemaphoreType.DMA,
        pltpu.SemaphoreType.DMA,
        pltpu.SemaphoreType.DMA,
        pltpu.SemaphoreType.DMA,
    ],
)(_gather_body)


def kernel(tokens, charges, aa_table, charge_table):
    tokens_t = tokens.astype(jnp.int32).T
    charges = charges.astype(jnp.int32).reshape(1, B)
    t3, tc, idx = _prep(tokens_t, charges)(
        jnp.asarray(_PE), aa_table, charge_table, tokens_t, charges)
    table = jnp.concatenate([t3.reshape(CB, DIM), tc], axis=0)
    idx2 = idx.reshape(NW, NCH, C)
    out = _sc_gather(table, idx2)
    return jnp.swapaxes(out.reshape(LP1, B, DIM), 0, 1)


# speedup vs baseline: 3.2531x; 1.0102x over previous
import functools

import numpy as np
import jax
import jax.numpy as jnp
from jax import lax
from jax.experimental import pallas as pl
from jax.experimental.pallas import tpu as pltpu
from jax.experimental.pallas import tpu_sc as plsc

B = 4096
L = 50
LP1 = L + 1
DIM = 512
VOCAB = 28
MAX_CHARGE = 10

VS = 32
CB = L * VS
TROWS = CB + 16

R = B * LP1
NC = 2
NS = 16
NW = NC * NS
RPW = R // NW
C = 96
NCH = RPW // C
NB = 2
NG = (NCH - 4) // 2


def _positional_encoding_np(length, d_model):
    pos = np.arange(length, dtype=np.float32)[:, None]
    i = np.arange(d_model // 2, dtype=np.float32)[None, :]
    angle = pos / np.power(10000.0, (2.0 * i) / d_model)
    pe = np.zeros((length, d_model), dtype=np.float32)
    pe[:, 0::2] = np.sin(angle)
    pe[:, 1::2] = np.cos(angle)
    return pe


_PE = _positional_encoding_np(L, DIM)


def _prep_body(pe_ref, aa_ref, ch_ref, tok_ref, chg_ref, t3_ref, tc_ref, idx_ref):
    aa = aa_ref[...]
    row = lax.broadcasted_iota(jnp.int32, (VOCAB, DIM), 0)
    aa_z = jnp.where(row == VOCAB - 1, 0.0, aa)
    aa_p = jnp.concatenate(
        [aa_z, jnp.zeros((VS - VOCAB, DIM), jnp.float32)], axis=0)
    pe = pe_ref[...]
    t3_ref[...] = pe[:, None, :] + aa_p[None, :, :]
    ch = ch_ref[...]
    tc_ref[...] = jnp.concatenate(
        [ch, jnp.zeros((TROWS - CB - MAX_CHARGE, DIM), jnp.float32)], axis=0)
    pos_off = lax.broadcasted_iota(jnp.int32, (L, B), 0) * VS
    idx_ref[...] = jnp.concatenate(
        [chg_ref[...] + CB, tok_ref[...] + pos_off], axis=0)


def _prep(tokens, charges):
    return pl.pallas_call(
        _prep_body,
        out_shape=[
            jax.ShapeDtypeStruct((L, VS, DIM), jnp.float32),
            jax.ShapeDtypeStruct((TROWS - CB, DIM), jnp.float32),
            jax.ShapeDtypeStruct((LP1, B), jnp.int32),
        ],
    )


def _gather_body(tab_hbm, idx_hbm, out_hbm, idx_v,
                 b0, b1, g0, g1, s0, s1):
    bufs, gsems, ssems = (b0, b1), (g0, g1), (s0, s1)
    wid = lax.axis_index("s") * NC + lax.axis_index("c")
    cbase = wid * NCH
    pltpu.sync_copy(idx_hbm.at[wid], idx_v)

    def g_start(k, p):
        pltpu.async_copy(tab_hbm.at[idx_v.at[k]], bufs[p], gsems[p])

    def g_wait(p):
        pltpu.make_async_copy(tab_hbm.at[pl.ds(0, C)], bufs[p], gsems[p]).wait()

    def s_start(k, p):
        pltpu.async_copy(bufs[p], out_hbm.at[pl.ds((cbase + k) * C, C)],
                         ssems[p])

    def s_wait(p):
        pltpu.make_async_copy(bufs[p], out_hbm.at[pl.ds(0, C)],
                              ssems[p]).wait()

    g_start(0, 0)
    g_wait(0); s_start(0, 0); g_start(1, 1)

    def group(g, carry):
        for a in range(2):
            k = 2 * g + 1 + a
            p = 1 - a
            g_wait(p)
            s_start(k, p)
            s_wait(1 - p)
            g_start(k + 1, 1 - p)
        return carry

    lax.fori_loop(0, NG, group, 0)

    k0 = NCH - 3
    g_wait(1); s_start(k0, 1); s_wait(0); g_start(k0 + 1, 0)
    g_wait(0); s_start(k0 + 1, 0); s_wait(1); g_start(k0 + 2, 1)
    g_wait(1); s_start(k0 + 2, 1); s_wait(0)
    s_wait(1)


_MESH = plsc.VectorSubcoreMesh(
    core_axis_name="c", subcore_axis_name="s", num_cores=NC, num_subcores=NS)

_sc_gather = functools.partial(
    pl.kernel,
    out_type=jax.ShapeDtypeStruct((R, DIM), jnp.float32),
    mesh=_MESH,
    scratch_types=[
        pltpu.VMEM((NCH, C), jnp.int32),
        pltpu.VMEM((C, DIM), jnp.float32),
        pltpu.VMEM((C, DIM), jnp.float32),
        pltpu.SemaphoreType.DMA,
        pltpu.SemaphoreType.DMA,
        pltpu.SemaphoreType.DMA,
        pltpu.SemaphoreType.DMA,
    ],
)(_gather_body)


def kernel(tokens, charges, aa_table, charge_table):
    tokens_t = tokens.astype(jnp.int32).T
    charges = charges.astype(jnp.int32).reshape(1, B)
    t3, tc, idx = _prep(tokens_t, charges)(
        jnp.asarray(_PE), aa_table, charge_table, tokens_t, charges)
    table = jnp.concatenate([t3.reshape(CB, DIM), tc], axis=0)
    idx2 = idx.reshape(NW, NCH, C)
    out = _sc_gather(table, idx2)
    return jnp.swapaxes(out.reshape(LP1, B, DIM), 0, 1)
